# baseline (speedup 1.0000x reference)
import functools

import jax
import jax.numpy as jnp
from jax import lax
from jax.experimental import pallas as pl
from jax.experimental.pallas import tpu as pltpu
from jax.experimental.pallas import tpu_sc as plsc

_K = 20
_D = 64
_NC = 2
_NS = 16
_NW = _NC * _NS
_ISL = 128
_GPR = 2
_RPR = _GPR * _ISL


@functools.lru_cache(maxsize=None)
def _make_sc_gather(B):
    PW = B // _NW
    NPW = PW * _K
    assert B % _NW == 0 and PW % _RPR == 0 and NPW % _RPR == 0

    mesh = plsc.VectorSubcoreMesh(core_axis_name="c", subcore_axis_name="s")

    @functools.partial(
        pl.kernel,
        out_type=(jax.ShapeDtypeStruct((B, _D), jnp.float32),
                  jax.ShapeDtypeStruct((B, _D), jnp.float32),
                  jax.ShapeDtypeStruct((B * _K, _D), jnp.float32)),
        mesh=mesh,
        compiler_params=pltpu.CompilerParams(
            needs_layout_passes=False, use_tc_tiling_on_sc=False),
        scratch_types=[
            pltpu.VMEM((PW,), jnp.int32),
            pltpu.VMEM((PW,), jnp.int32),
            pltpu.VMEM((NPW,), jnp.int32),
            pltpu.VMEM((2, _RPR, _D), jnp.float32),
            pltpu.SemaphoreType.DMA,
            pltpu.SemaphoreType.DMA,
            pltpu.SemaphoreType.DMA,
            pltpu.SemaphoreType.DMA,
        ],
    )
    def sgns_gather(centers_h, contexts_h, negs_h, inemb_h, outemb_h,
                    vc_h, uo_h, ne_h, cidx, oidx, nidx, buf,
                    gsemA, gsemB, wsemA, wsemB):
        wid = lax.axis_index("s") * _NC + lax.axis_index("c")
        gsems = (gsemA, gsemB)
        wsems = (wsemA, wsemB)

        pltpu.sync_copy(centers_h.at[pl.ds(wid * PW, PW)], cidx)
        pltpu.sync_copy(contexts_h.at[pl.ds(wid * PW, PW)], oidx)
        pltpu.sync_copy(negs_h.at[pl.ds(wid * NPW, NPW)], nidx)

        def gathers(idx, table_h, r, b):
            return [
                pltpu.make_async_copy(
                    table_h.at[idx.at[pl.ds(r * _RPR + j * _ISL, _ISL)]],
                    buf.at[b].at[pl.ds(j * _ISL, _ISL)], gsems[b])
                for j in range(_GPR)
            ]

        def phase(idx, table_h, out_h, base, nrounds):
            def start_g(r, b):
                for cp in gathers(idx, table_h, r, b):
                    cp.start()

            def wb(r, b):
                return pltpu.make_async_copy(
                    buf.at[b], out_h.at[pl.ds(base + r * _RPR, _RPR)],
                    wsems[b])

            start_g(0, 0)
            start_g(1, 1)

            def round_body(r, b):
                for cp in gathers(idx, table_h, r, b):
                    cp.wait()
                w = wb(r, b)
                w.start()

                @pl.when(r + 2 < nrounds)
                def _():
                    w.wait()
                    start_g(r + 2, b)

            if nrounds <= 4:
                for r in range(nrounds):
                    round_body(r, r % 2)
            else:
                def loop_body(i, _):
                    for b in range(2):
                        round_body(i * 2 + b, b)
                    return 0
                lax.fori_loop(0, nrounds // 2, loop_body, 0)

            wb(nrounds - 2, (nrounds - 2) % 2).wait()
            wb(nrounds - 1, (nrounds - 1) % 2).wait()

        phase(cidx, inemb_h, vc_h, wid * PW, PW // _RPR)
        phase(oidx, outemb_h, uo_h, wid * PW, PW // _RPR)
        phase(nidx, outemb_h, ne_h, wid * NPW, NPW // _RPR)

    return sgns_gather


@functools.lru_cache(maxsize=None)
def _make_tc_loss(B, BB):
    nblk = B // BB

    def body(vc_ref, uo_ref, ne_ref, out_ref):
        i = pl.program_id(0)
        vc = vc_ref[...]
        uo = uo_ref[...]
        pos = jnp.sum(vc * uo, axis=1)
        part = jnp.sum(-jnp.log(jax.nn.sigmoid(pos) + 1e-10))
        for k in range(_K):
            ns = jnp.sum(ne_ref[:, k, :] * vc, axis=1)
            part += jnp.sum(-jnp.log(jax.nn.sigmoid(-ns) + 1e-10))

        acc = (part / B).reshape(1, 1)

        @pl.when(i == 0)
        def _():
            out_ref[...] = acc

        @pl.when(i > 0)
        def _():
            out_ref[...] = out_ref[...] + acc

    return pl.pallas_call(
        body,
        grid=(nblk,),
        in_specs=[
            pl.BlockSpec((BB, _D), lambda i: (i, 0)),
            pl.BlockSpec((BB, _D), lambda i: (i, 0)),
            pl.BlockSpec((BB, _K, _D), lambda i: (i, 0, 0)),
        ],
        out_specs=pl.BlockSpec((1, 1), lambda i: (0, 0)),
        out_shape=jax.ShapeDtypeStruct((1, 1), jnp.float32),
    )


def kernel(centers, contexts, negs, in_embed, out_embed):
    B = centers.shape[0]
    c = centers.reshape(-1).astype(jnp.int32)
    o = contexts.reshape(-1).astype(jnp.int32)
    n = negs.reshape(-1).astype(jnp.int32)
    vc, uo, ne = _make_sc_gather(B)(c, o, n, in_embed, out_embed)
    loss = _make_tc_loss(B, 1024)(vc, uo, ne.reshape(B, _K, _D))
    return loss[0, 0]


# speedup vs baseline: 1.0817x; 1.0817x over previous
import functools

import jax
import jax.numpy as jnp
from jax import lax
from jax.experimental import pallas as pl
from jax.experimental.pallas import tpu as pltpu
from jax.experimental.pallas import tpu_sc as plsc

_K = 20
_D = 64
_NC = 2
_NS = 16
_NW = _NC * _NS
_ISL = 128
_GPR = 2
_RPR = _GPR * _ISL


@functools.lru_cache(maxsize=None)
def _make_sc_gather(B):
    PW = B // _NW
    NPW = PW * _K
    assert B % _NW == 0 and PW % _RPR == 0 and NPW % _RPR == 0

    mesh = plsc.VectorSubcoreMesh(core_axis_name="c", subcore_axis_name="s")

    @functools.partial(
        pl.kernel,
        out_type=(jax.ShapeDtypeStruct((B, _D), jnp.float32),
                  jax.ShapeDtypeStruct((B, _D), jnp.float32),
                  jax.ShapeDtypeStruct((B * _K, _D), jnp.float32)),
        mesh=mesh,
        compiler_params=pltpu.CompilerParams(
            needs_layout_passes=False, use_tc_tiling_on_sc=False),
        scratch_types=[
            pltpu.VMEM((PW,), jnp.int32),
            pltpu.VMEM((PW,), jnp.int32),
            pltpu.VMEM((NPW,), jnp.int32),
            pltpu.VMEM((2, _RPR, _D), jnp.float32),
            pltpu.SemaphoreType.DMA,
            pltpu.SemaphoreType.DMA,
            pltpu.SemaphoreType.DMA,
            pltpu.SemaphoreType.DMA,
        ],
    )
    def sgns_gather(centers_h, contexts_h, negs_h, inemb_h, outemb_h,
                    vc_h, uo_h, ne_h, cidx, oidx, nidx, buf,
                    gsemA, gsemB, wsemA, wsemB):
        wid = lax.axis_index("s") * _NC + lax.axis_index("c")
        gsems = (gsemA, gsemB)
        wsems = (wsemA, wsemB)

        pltpu.sync_copy(centers_h.at[pl.ds(wid * PW, PW)], cidx)
        pltpu.sync_copy(contexts_h.at[pl.ds(wid * PW, PW)], oidx)
        pltpu.sync_copy(negs_h.at[pl.ds(wid * NPW, NPW)], nidx)

        def gathers(idx, table_h, r, b):
            return [
                pltpu.make_async_copy(
                    table_h.at[idx.at[pl.ds(r * _RPR + j * _ISL, _ISL)]],
                    buf.at[b].at[pl.ds(j * _ISL, _ISL)], gsems[b])
                for j in range(_GPR)
            ]

        def phase(idx, table_h, out_h, base, nrounds):
            def start_g(r, b):
                for cp in gathers(idx, table_h, r, b):
                    cp.start()

            def wb(r, b):
                return pltpu.make_async_copy(
                    buf.at[b], out_h.at[pl.ds(base + r * _RPR, _RPR)],
                    wsems[b])

            start_g(0, 0)
            start_g(1, 1)

            def round_body(r, b):
                for cp in gathers(idx, table_h, r, b):
                    cp.wait()
                w = wb(r, b)
                w.start()

                @pl.when(r + 2 < nrounds)
                def _():
                    w.wait()
                    start_g(r + 2, b)

            if nrounds <= 4:
                for r in range(nrounds):
                    round_body(r, r % 2)
            else:
                def loop_body(i, _):
                    for b in range(2):
                        round_body(i * 2 + b, b)
                    return 0
                lax.fori_loop(0, nrounds // 2, loop_body, 0)

            wb(nrounds - 2, (nrounds - 2) % 2).wait()
            wb(nrounds - 1, (nrounds - 1) % 2).wait()

        phase(cidx, inemb_h, vc_h, wid * PW, PW // _RPR)
        phase(oidx, outemb_h, uo_h, wid * PW, PW // _RPR)
        phase(nidx, outemb_h, ne_h, wid * NPW, NPW // _RPR)

    return sgns_gather


@functools.lru_cache(maxsize=None)
def _make_tc_loss(B, BB):
    nblk = B // BB

    def body(vc_ref, uo_ref, ne_ref, out_ref):
        i = pl.program_id(0)
        k = pl.program_id(1)
        vc = vc_ref[...]
        ne = ne_ref[0]
        ns = jnp.sum(ne * vc, axis=1)
        part = jnp.sum(-jnp.log(jax.nn.sigmoid(-ns) + 1e-10))

        @pl.when(jnp.logical_and(i == 0, k == 0))
        def _():
            out_ref[...] = jnp.zeros_like(out_ref)

        @pl.when(k == 0)
        def _():
            pos = jnp.sum(vc * uo_ref[...], axis=1)
            out_ref[...] = out_ref[...] + (
                jnp.sum(-jnp.log(jax.nn.sigmoid(pos) + 1e-10)) / B
            ).reshape(1, 1)

        out_ref[...] = out_ref[...] + (part / B).reshape(1, 1)

    return pl.pallas_call(
        body,
        grid=(nblk, _K),
        in_specs=[
            pl.BlockSpec((BB, _D), lambda i, k: (i, 0)),
            pl.BlockSpec((BB, _D), lambda i, k: (i, 0)),
            pl.BlockSpec((1, BB, _D), lambda i, k: (k, i, 0)),
        ],
        out_specs=pl.BlockSpec((1, 1), lambda i, k: (0, 0)),
        out_shape=jax.ShapeDtypeStruct((1, 1), jnp.float32),
    )


def kernel(centers, contexts, negs, in_embed, out_embed):
    B = centers.shape[0]
    c = centers.reshape(-1).astype(jnp.int32)
    o = contexts.reshape(-1).astype(jnp.int32)
    n = negs.astype(jnp.int32).T.reshape(-1)
    vc, uo, ne = _make_sc_gather(B)(c, o, n, in_embed, out_embed)
    loss = _make_tc_loss(B, 4096)(vc, uo, ne.reshape(_K, B, _D))
    return loss[0, 0]


# speedup vs baseline: 1.1058x; 1.0223x over previous
import functools

import jax
import jax.numpy as jnp
from jax import lax
from jax.experimental import pallas as pl
from jax.experimental.pallas import tpu as pltpu
from jax.experimental.pallas import tpu_sc as plsc

_K = 20
_D = 64
_NC = 2
_NS = 16
_NW = _NC * _NS
_ISL = 128
_GPR = 2
_RPR = _GPR * _ISL


@functools.lru_cache(maxsize=None)
def _make_sc_gather(B):
    PW = B // _NW
    NPW = PW * _K
    assert B % _NW == 0 and PW % _RPR == 0 and NPW % _RPR == 0

    mesh = plsc.VectorSubcoreMesh(core_axis_name="c", subcore_axis_name="s")

    @functools.partial(
        pl.kernel,
        out_type=(jax.ShapeDtypeStruct((B, _D), jnp.float32),
                  jax.ShapeDtypeStruct((B, _D), jnp.float32),
                  jax.ShapeDtypeStruct((B * _K, _D), jnp.float32)),
        mesh=mesh,
        compiler_params=pltpu.CompilerParams(
            needs_layout_passes=False, use_tc_tiling_on_sc=False),
        scratch_types=[
            pltpu.VMEM((PW,), jnp.int32),
            pltpu.VMEM((PW,), jnp.int32),
            pltpu.VMEM((NPW,), jnp.int32),
            pltpu.VMEM((2, _RPR, _D), jnp.float32),
            pltpu.SemaphoreType.DMA,
            pltpu.SemaphoreType.DMA,
            pltpu.SemaphoreType.DMA,
            pltpu.SemaphoreType.DMA,
        ],
    )
    def sgns_gather(centers_h, contexts_h, negs_h, inemb_h, outemb_h,
                    vc_h, uo_h, ne_h, cidx, oidx, nidx, buf,
                    gsemA, gsemB, wsemA, wsemB):
        wid = lax.axis_index("s") * _NC + lax.axis_index("c")
        gsems = (gsemA, gsemB)
        wsems = (wsemA, wsemB)

        pltpu.sync_copy(centers_h.at[pl.ds(wid * PW, PW)], cidx)
        pltpu.sync_copy(contexts_h.at[pl.ds(wid * PW, PW)], oidx)
        pltpu.sync_copy(negs_h.at[pl.ds(wid * NPW, NPW)], nidx)

        def gathers(idx, table_h, r, b):
            return [
                pltpu.make_async_copy(
                    table_h.at[idx.at[pl.ds(r * _RPR + j * _ISL, _ISL)]],
                    buf.at[b].at[pl.ds(j * _ISL, _ISL)], gsems[b])
                for j in range(_GPR)
            ]

        def phase(idx, table_h, out_h, base, nrounds):
            def start_g(r, b):
                for cp in gathers(idx, table_h, r, b):
                    cp.start()

            def wb(r, b):
                return pltpu.make_async_copy(
                    buf.at[b], out_h.at[pl.ds(base + r * _RPR, _RPR)],
                    wsems[b])

            start_g(0, 0)
            start_g(1, 1)

            def round_body(r, b):
                for cp in gathers(idx, table_h, r, b):
                    cp.wait()
                w = wb(r, b)
                w.start()

                @pl.when(r + 2 < nrounds)
                def _():
                    w.wait()
                    start_g(r + 2, b)

            if nrounds <= 4:
                for r in range(nrounds):
                    round_body(r, r % 2)
            else:
                def loop_body(i, _):
                    for b in range(2):
                        round_body(i * 2 + b, b)
                    return 0
                lax.fori_loop(0, nrounds // 2, loop_body, 0)

            wb(nrounds - 2, (nrounds - 2) % 2).wait()
            wb(nrounds - 1, (nrounds - 1) % 2).wait()

        phase(cidx, inemb_h, vc_h, wid * PW, PW // _RPR)
        phase(oidx, outemb_h, uo_h, wid * PW, PW // _RPR)
        phase(nidx, outemb_h, ne_h, wid * NPW, NPW // _RPR)

    return sgns_gather


@functools.lru_cache(maxsize=None)
def _make_tc_loss(B, BB):
    nblk = B // BB

    def body(vc_ref, uo_ref, ne_ref, out_ref):
        i = pl.program_id(0)
        vc = vc_ref[...]
        pos = jnp.sum(vc * uo_ref[...], axis=1)
        acc = jnp.sum(-jnp.log(jax.nn.sigmoid(pos) + 1e-10))
        ns = jnp.sum(ne_ref[...] * vc[None], axis=2)
        acc = acc + jnp.sum(-jnp.log(jax.nn.sigmoid(-ns) + 1e-10))

        @pl.when(i == 0)
        def _():
            out_ref[...] = jnp.zeros_like(out_ref)

        out_ref[...] = out_ref[...] + (acc / B).reshape(1, 1)

    return pl.pallas_call(
        body,
        grid=(nblk,),
        in_specs=[
            pl.BlockSpec((BB, _D), lambda i: (i, 0)),
            pl.BlockSpec((BB, _D), lambda i: (i, 0)),
            pl.BlockSpec((_K, BB, _D), lambda i: (0, i, 0)),
        ],
        out_specs=pl.BlockSpec((1, 1), lambda i: (0, 0)),
        out_shape=jax.ShapeDtypeStruct((1, 1), jnp.float32),
    )


def kernel(centers, contexts, negs, in_embed, out_embed):
    B = centers.shape[0]
    c = centers.reshape(-1).astype(jnp.int32)
    o = contexts.reshape(-1).astype(jnp.int32)
    n = negs.astype(jnp.int32).T.reshape(-1)
    vc, uo, ne = _make_sc_gather(B)(c, o, n, in_embed, out_embed)
    loss = _make_tc_loss(B, 1024)(vc, uo, ne.reshape(_K, B, _D))
    return loss[0, 0]


# speedup vs baseline: 1.1078x; 1.0017x over previous
import functools

import jax
import jax.numpy as jnp
from jax import lax
from jax.experimental import pallas as pl
from jax.experimental.pallas import tpu as pltpu
from jax.experimental.pallas import tpu_sc as plsc

_K = 20
_D = 64
_NC = 2
_NS = 16
_NW = _NC * _NS
_ISL = 128
_GPR = 2
_RPR = _GPR * _ISL


@functools.lru_cache(maxsize=None)
def _make_sc_gather(B):
    PW = B // _NW
    NPW = PW * _K
    assert B % _NW == 0 and PW % _RPR == 0 and NPW % _RPR == 0

    mesh = plsc.VectorSubcoreMesh(core_axis_name="c", subcore_axis_name="s")

    @functools.partial(
        pl.kernel,
        out_type=(jax.ShapeDtypeStruct((B, _D), jnp.float32),
                  jax.ShapeDtypeStruct((B, _D), jnp.float32),
                  jax.ShapeDtypeStruct((B * _K, _D), jnp.float32)),
        mesh=mesh,
        compiler_params=pltpu.CompilerParams(
            needs_layout_passes=True, use_tc_tiling_on_sc=False),
        scratch_types=[
            pltpu.VMEM((PW,), jnp.int32),
            pltpu.VMEM((PW,), jnp.int32),
            pltpu.VMEM((NPW,), jnp.int32),
            pltpu.VMEM((2, _RPR, _D), jnp.float32),
            pltpu.SemaphoreType.DMA,
            pltpu.SemaphoreType.DMA,
            pltpu.SemaphoreType.DMA,
            pltpu.SemaphoreType.DMA,
        ],
    )
    def sgns_gather(centers_h, contexts_h, negs_h, inemb_h, outemb_h,
                    vc_h, uo_h, ne_h, cidx, oidx, nidx, buf,
                    gsemA, gsemB, wsemA, wsemB):
        wid = lax.axis_index("s") * _NC + lax.axis_index("c")
        gsems = (gsemA, gsemB)
        wsems = (wsemA, wsemB)

        pltpu.sync_copy(centers_h.at[pl.ds(wid * PW, PW)], cidx)
        pltpu.sync_copy(contexts_h.at[pl.ds(wid * PW, PW)], oidx)
        pltpu.sync_copy(negs_h.at[pl.ds(wid * NPW, NPW)], nidx)

        def gathers(idx, table_h, r, b):
            return [
                pltpu.make_async_copy(
                    table_h.at[idx.at[pl.ds(r * _RPR + j * _ISL, _ISL)]],
                    buf.at[b].at[pl.ds(j * _ISL, _ISL)], gsems[b])
                for j in range(_GPR)
            ]

        def phase(idx, table_h, out_h, base, nrounds):
            def start_g(r, b):
                for cp in gathers(idx, table_h, r, b):
                    cp.start()

            def wb(r, b):
                return pltpu.make_async_copy(
                    buf.at[b], out_h.at[pl.ds(base + r * _RPR, _RPR)],
                    wsems[b])

            start_g(0, 0)
            start_g(1, 1)

            def round_body(r, b):
                for cp in gathers(idx, table_h, r, b):
                    cp.wait()
                w = wb(r, b)
                w.start()

                @pl.when(r + 2 < nrounds)
                def _():
                    w.wait()
                    start_g(r + 2, b)

            if nrounds <= 4:
                for r in range(nrounds):
                    round_body(r, r % 2)
            else:
                def loop_body(i, _):
                    for b in range(2):
                        round_body(i * 2 + b, b)
                    return 0
                lax.fori_loop(0, nrounds // 2, loop_body, 0)

            wb(nrounds - 2, (nrounds - 2) % 2).wait()
            wb(nrounds - 1, (nrounds - 1) % 2).wait()

        phase(cidx, inemb_h, vc_h, wid * PW, PW // _RPR)
        phase(oidx, outemb_h, uo_h, wid * PW, PW // _RPR)
        phase(nidx, outemb_h, ne_h, wid * NPW, NPW // _RPR)

    return sgns_gather


@functools.lru_cache(maxsize=None)
def _make_tc_loss(B, BB):
    nblk = B // BB

    def body(vc_ref, uo_ref, ne_ref, out_ref):
        i = pl.program_id(0)
        vc = vc_ref[...]
        pos = jnp.sum(vc * uo_ref[...], axis=1)
        acc = jnp.sum(-jnp.log(jax.nn.sigmoid(pos) + 1e-10))
        ns = jnp.sum(ne_ref[...] * vc[None], axis=2)
        acc = acc + jnp.sum(-jnp.log(jax.nn.sigmoid(-ns) + 1e-10))

        @pl.when(i == 0)
        def _():
            out_ref[...] = jnp.zeros_like(out_ref)

        out_ref[...] = out_ref[...] + (acc / B).reshape(1, 1)

    return pl.pallas_call(
        body,
        grid=(nblk,),
        in_specs=[
            pl.BlockSpec((BB, _D), lambda i: (i, 0)),
            pl.BlockSpec((BB, _D), lambda i: (i, 0)),
            pl.BlockSpec((_K, BB, _D), lambda i: (0, i, 0)),
        ],
        out_specs=pl.BlockSpec((1, 1), lambda i: (0, 0)),
        out_shape=jax.ShapeDtypeStruct((1, 1), jnp.float32),
    )


def kernel(centers, contexts, negs, in_embed, out_embed):
    B = centers.shape[0]
    c = centers.reshape(-1).astype(jnp.int32)
    o = contexts.reshape(-1).astype(jnp.int32)
    n = negs.astype(jnp.int32).T.reshape(-1)
    vc, uo, ne = _make_sc_gather(B)(c, o, n, in_embed, out_embed)
    loss = _make_tc_loss(B, 1024)(vc, uo, ne.reshape(_K, B, _D))
    return loss[0, 0]
